# baseline (speedup 1.0000x reference)
import jax
import jax.numpy as jnp
from jax.experimental import pallas as pl

OUT_DIM = 1000
N = 16384
BLOCK_ROWS = 1024
BLOCK_COLS = 128
NUM_BLOCKS = N // BLOCK_ROWS
NUM_CBLOCKS = (OUT_DIM + BLOCK_COLS - 1) // BLOCK_COLS


def _onehot_block(idx_ref, out_ref):
    j = pl.program_id(1)
    idx = idx_ref[0, 0, :]
    cols = j * BLOCK_COLS + jax.lax.broadcasted_iota(
        jnp.int32, (BLOCK_ROWS, BLOCK_COLS), 1)
    out_ref[...] = (cols == idx[:, None]).astype(jnp.float32)


def kernel(inputs):
    idx = inputs.astype(jnp.int32).reshape(NUM_BLOCKS, 1, BLOCK_ROWS)
    return pl.pallas_call(
        _onehot_block,
        grid=(NUM_BLOCKS, NUM_CBLOCKS),
        in_specs=[pl.BlockSpec((1, 1, BLOCK_ROWS), lambda i, j: (i, 0, 0))],
        out_specs=pl.BlockSpec((BLOCK_ROWS, BLOCK_COLS), lambda i, j: (i, j)),
        out_shape=jax.ShapeDtypeStruct((N, OUT_DIM), jnp.float32),
    )(idx)


# speedup vs baseline: 1.4908x; 1.4908x over previous
import jax
import jax.numpy as jnp
from jax.experimental import pallas as pl

OUT_DIM = 1000
PAD_DIM = 1024
N = 16384
BLOCK_ROWS = 1024
NUM_BLOCKS = N // BLOCK_ROWS


def _onehot_block(idx_ref, out_ref):
    idx = idx_ref[0, 0, :]
    cols = jax.lax.broadcasted_iota(jnp.int32, (BLOCK_ROWS, PAD_DIM), 1)
    out_ref[...] = (cols == idx[:, None]).astype(jnp.float32)


def kernel(inputs):
    idx = inputs.astype(jnp.int32).reshape(NUM_BLOCKS, 1, BLOCK_ROWS)
    padded = pl.pallas_call(
        _onehot_block,
        grid=(NUM_BLOCKS,),
        in_specs=[pl.BlockSpec((1, 1, BLOCK_ROWS), lambda i: (i, 0, 0))],
        out_specs=pl.BlockSpec((BLOCK_ROWS, PAD_DIM), lambda i: (i, 0)),
        out_shape=jax.ShapeDtypeStruct((N, PAD_DIM), jnp.float32),
    )(idx)
    return padded[:, :OUT_DIM]
